# baseline (speedup 1.0000x reference)
import functools

import jax
import jax.numpy as jnp
from jax import lax
from jax.experimental import pallas as pl
from jax.experimental.pallas import tpu as pltpu
from jax.experimental.pallas import tpu_sc as plsc

D = 64
NC = 2
NS = 16
NW = NC * NS
CHUNK = 128
GRP = 16
NQ = D // GRP
W2 = 12800


def _rsqrt_nr(s):
    i = lax.bitcast_convert_type(s, jnp.int32)
    i = jnp.int32(0x5F3759DF) - lax.shift_right_logical(i, 1)
    y = lax.bitcast_convert_type(i, jnp.float32)
    half_s = jnp.float32(0.5) * s
    for _ in range(3):
        y = y * (jnp.float32(1.5) - half_s * y * y)
    return y


def _transpose_table(x1t):
    n = x1t.shape[1]
    n_blocks = (n + W2 - 1) // W2
    grid = (n_blocks + 1) // 2
    last = n_blocks - 1

    def body(a_ref, b_ref, o_ref):
        o_ref[:, :D] = a_ref[...].T
        o_ref[:, D:] = b_ref[...].T

    return pl.pallas_call(
        body,
        grid=(grid,),
        in_specs=[
            pl.BlockSpec((D, W2), lambda i: (0, jnp.minimum(2 * i, last))),
            pl.BlockSpec((D, W2), lambda i: (0, jnp.minimum(2 * i + 1, last))),
        ],
        out_specs=pl.BlockSpec((W2, 2 * D), lambda i: (i, 0)),
        out_shape=jax.ShapeDtypeStruct((grid * W2, 2 * D), jnp.float32),
    )(x1t, x1t)


@functools.partial(jax.jit, static_argnums=(3,))
def _gather_normalize(table2, pidx3d, hpar3d, b_each):
    n_chunks = pidx3d.shape[1]
    n_pairs = n_chunks // 2
    w_per_arr = NW // 4
    mesh = plsc.VectorSubcoreMesh(core_axis_name="c", subcore_axis_name="s")
    out_sds = jax.ShapeDtypeStruct((b_each, D), jnp.float32)

    @functools.partial(
        pl.kernel,
        mesh=mesh,
        out_type=(out_sds, out_sds, out_sds, out_sds),
        compiler_params=pltpu.CompilerParams(use_tc_tiling_on_sc=True),
        scratch_types=[
            pltpu.VMEM((n_chunks, CHUNK), jnp.int32),
            pltpu.VMEM((n_chunks, CHUNK), jnp.int32),
            pltpu.VMEM((CHUNK, 2 * D), jnp.float32),
            pltpu.VMEM((CHUNK, 2 * D), jnp.float32),
            pltpu.VMEM((CHUNK, D), jnp.float32),
            pltpu.SemaphoreType.DMA,
            pltpu.SemaphoreType.DMA,
        ],
    )
    def k(table_hbm, pidx_hbm, hpar_hbm, o0, o1, o2, o3,
          pidx_v, hpar_v, b0, b1, ob, g0, g1):
        wid = lax.axis_index("s") * NC + lax.axis_index("c")
        aid = wid // w_per_arr
        arow = (wid % w_per_arr) * (n_chunks * CHUNK)
        pltpu.sync_copy(pidx_hbm.at[wid], pidx_v)
        pltpu.sync_copy(hpar_hbm.at[wid], hpar_v)

        lane = lax.iota(jnp.int32, GRP)
        perms = [lax.bitwise_xor(lane, jnp.int32(off)) for off in (8, 4, 2, 1)]

        def fire(c, buf, sem):
            def rows(g, carry):
                iv = pidx_v[c, pl.ds(g * GRP, GRP)]
                for u in range(GRP):
                    pltpu.async_copy(
                        table_hbm.at[pl.ds(iv[u], 1)],
                        buf.at[pl.ds(g * GRP + u, 1)],
                        sem,
                    )
                return carry

            lax.fori_loop(0, CHUNK // GRP, rows, 0)

        def drain(buf, sem):
            pltpu.make_async_copy(
                table_hbm.at[pl.ds(0, CHUNK)], buf, sem
            ).wait()

        def normalize(buf, c):
            def rows_body(i, carry):
                h16 = hpar_v[c, pl.ds(i * GRP, GRP)]
                for u in range(GRP):
                    r = i * GRP + u
                    off = h16[u] * D
                    vs = [buf[r, pl.ds(off + q * GRP, GRP)] for q in range(NQ)]
                    s = jnp.zeros((GRP,), jnp.float32)
                    for v in vs:
                        s = s + v * v
                    for p in perms:
                        s = s + jnp.take_along_axis(
                            s, p, axis=0, mode="promise_in_bounds"
                        )
                    rs = _rsqrt_nr(jnp.maximum(s, jnp.float32(1e-24)))
                    for q, v in enumerate(vs):
                        ob[r, pl.ds(q * GRP, GRP)] = v * rs
                return carry

            lax.fori_loop(0, CHUNK // GRP, rows_body, 0)

        def write_out(c):
            dst_row = arow + c * CHUNK
            for a, out in enumerate((o0, o1, o2, o3)):
                @pl.when(aid == a)
                def _():
                    pltpu.sync_copy(ob, out.at[pl.ds(dst_row, CHUNK)])

        fire(0, b0, g0)

        def body(i, carry):
            c0 = 2 * i
            c1 = 2 * i + 1
            drain(b0, g0)
            fire(c1, b1, g1)
            normalize(b0, c0)
            write_out(c0)
            drain(b1, g1)

            @pl.when(i + 1 < n_pairs)
            def _():
                fire(c0 + 2, b0, g0)

            normalize(b1, c1)
            write_out(c1)
            return carry

        lax.fori_loop(0, n_pairs, body, 0)

    return k(table2, pidx3d, hpar3d)


def kernel(x1, adj, pos_src, pos_dst, neg_src, neg_dst):
    del adj
    b_each = pos_src.shape[0]
    idx = jnp.concatenate([
        pos_src.astype(jnp.int32), pos_dst.astype(jnp.int32),
        neg_src.astype(jnp.int32), neg_dst.astype(jnp.int32),
    ])
    per_w = (4 * b_each) // NW
    kblk = idx // W2
    j = idx - kblk * W2
    pidx3d = ((kblk >> 1) * W2 + j).reshape(NW, per_w // CHUNK, CHUNK)
    hpar3d = (kblk & 1).reshape(NW, per_w // CHUNK, CHUNK)
    table2 = _transpose_table(x1.T)
    return _gather_normalize(table2, pidx3d, hpar3d, b_each)


# speedup vs baseline: 1.0890x; 1.0890x over previous
import functools

import jax
import jax.numpy as jnp
from jax import lax
from jax.experimental import pallas as pl
from jax.experimental.pallas import tpu as pltpu
from jax.experimental.pallas import tpu_sc as plsc

D = 64
NC = 2
NS = 16
NW = NC * NS
CHUNK = 128
GRP = 16
NQ = D // GRP
W2 = 12800


def _rsqrt_nr(s):
    i = lax.bitcast_convert_type(s, jnp.int32)
    i = jnp.int32(0x5F3759DF) - lax.shift_right_logical(i, 1)
    y = lax.bitcast_convert_type(i, jnp.float32)
    half_s = jnp.float32(0.5) * s
    for _ in range(3):
        y = y * (jnp.float32(1.5) - half_s * y * y)
    return y


def _transpose_table(x1t):
    n = x1t.shape[1]
    n_blocks = (n + W2 - 1) // W2
    grid = (n_blocks + 1) // 2
    last = n_blocks - 1

    def body(a_ref, b_ref, o_ref):
        o_ref[:, :D] = a_ref[...].T
        o_ref[:, D:] = b_ref[...].T

    return pl.pallas_call(
        body,
        grid=(grid,),
        in_specs=[
            pl.BlockSpec((D, W2), lambda i: (0, jnp.minimum(2 * i, last))),
            pl.BlockSpec((D, W2), lambda i: (0, jnp.minimum(2 * i + 1, last))),
        ],
        out_specs=pl.BlockSpec((W2, 2 * D), lambda i: (i, 0)),
        out_shape=jax.ShapeDtypeStruct((grid * W2, 2 * D), jnp.float32),
    )(x1t, x1t)


@functools.partial(jax.jit, static_argnums=(2,))
def _gather_normalize(tflat, woff3d, b_each):
    n_chunks = woff3d.shape[1]
    n_pairs = n_chunks // 2
    w_per_arr = NW // 4
    mesh = plsc.VectorSubcoreMesh(core_axis_name="c", subcore_axis_name="s")
    out_sds = jax.ShapeDtypeStruct((b_each // 2, 2 * D), jnp.float32)

    @functools.partial(
        pl.kernel,
        mesh=mesh,
        out_type=(out_sds, out_sds, out_sds, out_sds),
        compiler_params=pltpu.CompilerParams(use_tc_tiling_on_sc=False),
        scratch_types=[
            pltpu.VMEM((n_chunks, CHUNK), jnp.int32),
            pltpu.VMEM((CHUNK * D,), jnp.float32),
            pltpu.VMEM((CHUNK * D,), jnp.float32),
            pltpu.VMEM((CHUNK // 2, 2 * D), jnp.float32),
            pltpu.SemaphoreType.DMA,
            pltpu.SemaphoreType.DMA,
        ],
    )
    def k(tflat_hbm, woff_hbm, o0, o1, o2, o3, woff_v, b0, b1, ob, g0, g1):
        wid = lax.axis_index("s") * NC + lax.axis_index("c")
        aid = wid // w_per_arr
        arow = (wid % w_per_arr) * (n_chunks * CHUNK)
        pltpu.sync_copy(woff_hbm.at[wid], woff_v)

        lane = lax.iota(jnp.int32, GRP)
        perms = [lax.bitwise_xor(lane, jnp.int32(off)) for off in (8, 4, 2, 1)]

        def fire(c, buf, sem):
            def rows(g, carry):
                wv = woff_v[c, pl.ds(g * GRP, GRP)]
                for u in range(GRP):
                    pltpu.async_copy(
                        tflat_hbm.at[pl.ds(pl.multiple_of(wv[u], D), D)],
                        buf.at[pl.ds((g * GRP + u) * D, D)],
                        sem,
                    )
                return carry

            lax.fori_loop(0, CHUNK // GRP, rows, 0)

        def drain(buf, sem):
            pltpu.make_async_copy(
                tflat_hbm.at[pl.ds(0, CHUNK * D)], buf, sem
            ).wait()

        def normalize(buf):
            def rows_body(i, carry):
                for u in range(4):
                    r = i * 4 + u
                    vs = [
                        buf[pl.ds(r * D + q * GRP, GRP)] for q in range(NQ)
                    ]
                    s = jnp.zeros((GRP,), jnp.float32)
                    for v in vs:
                        s = s + v * v
                    for p in perms:
                        s = s + jnp.take_along_axis(
                            s, p, axis=0, mode="promise_in_bounds"
                        )
                    rs = _rsqrt_nr(jnp.maximum(s, jnp.float32(1e-24)))
                    base = (r % 2) * D
                    for q, v in enumerate(vs):
                        ob[r // 2, pl.ds(base + q * GRP, GRP)] = v * rs
                return carry

            lax.fori_loop(0, CHUNK // 4, rows_body, 0)

        def write_out(c):
            dst_row = (arow + c * CHUNK) // 2
            for a, out in enumerate((o0, o1, o2, o3)):
                @pl.when(aid == a)
                def _():
                    pltpu.sync_copy(ob, out.at[pl.ds(dst_row, CHUNK // 2)])

        fire(0, b0, g0)

        def body(i, carry):
            c0 = 2 * i
            c1 = 2 * i + 1
            drain(b0, g0)
            fire(c1, b1, g1)
            normalize(b0)
            write_out(c0)
            drain(b1, g1)

            @pl.when(i + 1 < n_pairs)
            def _():
                fire(c0 + 2, b0, g0)

            normalize(b1)
            write_out(c1)
            return carry

        lax.fori_loop(0, n_pairs, body, 0)

    return k(tflat, woff3d)


def kernel(x1, adj, pos_src, pos_dst, neg_src, neg_dst):
    del adj
    b_each = pos_src.shape[0]
    idx = jnp.concatenate([
        pos_src.astype(jnp.int32), pos_dst.astype(jnp.int32),
        neg_src.astype(jnp.int32), neg_dst.astype(jnp.int32),
    ])
    per_w = (4 * b_each) // NW
    kblk = idx // W2
    j = idx - kblk * W2
    woff = ((kblk >> 1) * W2 + j) * (2 * D) + (kblk & 1) * D
    woff3d = woff.reshape(NW, per_w // CHUNK, CHUNK)
    table2 = _transpose_table(x1.T)
    tflat = table2.reshape(-1)
    outs = _gather_normalize(tflat, woff3d, b_each)
    return tuple(o.reshape(b_each, D) for o in outs)
